# Initial kernel scaffold; baseline (speedup 1.0000x reference)
import jax
import jax.numpy as jnp
from jax.experimental import pallas as pl


def kernel(X, log_A, log_pi, means, log_vars):
    raise NotImplementedError("write your pallas kernel here")


# speedup vs baseline: 4.9154x; 4.9154x over previous
import functools
import math

import jax
import jax.numpy as jnp
from jax.experimental import pallas as pl
from jax.experimental.pallas import tpu as pltpu

_B = 16
_L = 2048
_T = _B * _L
_D = 32
_K = 16
_LOG2PI = math.log(2.0 * math.pi)
_LN2 = math.log(2.0)


def _hmm_body(xt_ref, logA_ref, logpi_ref, means_ref, logvars_ref,
              out_ref, eb_ref, m_ref):
    x = xt_ref[:]
    lv = logvars_ref[:]
    iv = jnp.exp(-lv)
    w = means_ref[:] * iv
    cst = jnp.sum(means_ref[:] * w + lv, axis=1, keepdims=True) + _D * _LOG2PI
    dn = (((1,), (1,)), ((), ()))
    q = (jax.lax.dot_general(x * x, iv, dn, preferred_element_type=jnp.float32)
         - 2.0 * jax.lax.dot_general(x, w, dn, preferred_element_type=jnp.float32))
    logb = -0.5 * (q + cst.reshape(1, _K))
    m = jnp.max(logb, axis=1, keepdims=True)
    eb_ref[:] = jnp.exp(logb - m)
    m_ref[:] = m

    expA = jnp.exp(logA_ref[:])
    pi = jnp.exp(logpi_ref[:])

    def renorm(ah, eacc):
        c = jnp.sum(ah, axis=1, keepdims=True)
        ebits = jax.lax.bitcast_convert_type(c, jnp.int32) >> 23
        fac = jax.lax.bitcast_convert_type((254 - ebits) << 23, jnp.float32)
        return ah * fac, eacc + (ebits - 127)

    alpha0 = pi * eb_ref[0:_B, :]
    alpha, eacc = renorm(alpha0, jnp.zeros((_B, 1), jnp.int32))
    msum = m_ref[0:_B, :]

    def step(t, carry):
        alpha, eacc, msum = carry
        ebt = eb_ref[pl.ds(t * _B, _B), :]
        mt = m_ref[pl.ds(t * _B, _B), :]
        ah = jax.lax.dot_general(alpha, expA, (((1,), (0,)), ((), ())),
                                 preferred_element_type=jnp.float32) * ebt
        alpha, eacc = renorm(ah, eacc)
        return alpha, eacc, msum + mt

    alpha, eacc, msum = jax.lax.fori_loop(1, _L, step, (alpha, eacc, msum))
    s = jnp.sum(alpha, axis=1, keepdims=True)
    out_ref[:] = msum + eacc.astype(jnp.float32) * _LN2 + jnp.log(s)


@functools.partial(jax.jit, static_argnames=())
def kernel(X, log_A, log_pi, means, log_vars):
    xt = X.reshape(_B, _L, _D).swapaxes(0, 1).reshape(_T, _D)
    out = pl.pallas_call(
        _hmm_body,
        out_shape=jax.ShapeDtypeStruct((_B, 1), jnp.float32),
        scratch_shapes=[
            pltpu.VMEM((_T, _K), jnp.float32),
            pltpu.VMEM((_T, 1), jnp.float32),
        ],
    )(xt, log_A, log_pi.reshape(1, _K), means, log_vars)
    return out.reshape(_B)


# speedup vs baseline: 6.9825x; 1.4205x over previous
import functools
import math

import jax
import jax.numpy as jnp
from jax.experimental import pallas as pl
from jax.experimental.pallas import tpu as pltpu

_B = 16
_L = 2048
_T = _B * _L
_D = 32
_K = 16
_LOG2PI = math.log(2.0 * math.pi)
_LN2 = math.log(2.0)
_NB = 4
_NBLK = (_L - 1) // _NB
_TAIL0 = 1 + _NBLK * _NB


def _hmm_body(xt_ref, logA_ref, logpi_ref, means_ref, logvars_ref,
              out_ref, eb_ref, m_ref):
    x = xt_ref[:]
    lv = logvars_ref[:]
    iv = jnp.exp(-lv)
    w = means_ref[:] * iv
    cst = jnp.sum(means_ref[:] * w + lv, axis=1, keepdims=True) + _D * _LOG2PI
    dn = (((1,), (1,)), ((), ()))
    q = (jax.lax.dot_general(x * x, iv, dn, preferred_element_type=jnp.float32)
         - 2.0 * jax.lax.dot_general(x, w, dn, preferred_element_type=jnp.float32))
    logb = (-0.5 * (q + cst.reshape(1, _K))).reshape(_L, _B, _K)
    m3 = jnp.max(logb, axis=2, keepdims=True)
    eb_ref[:] = jnp.exp(logb - m3)
    m_ref[:] = jnp.sum(m3, axis=0)

    expA = jnp.exp(logA_ref[:])
    pi = jnp.exp(logpi_ref[:])
    dnm = (((1,), (0,)), ((), ()))

    def renorm(ah, eacc):
        c = jnp.sum(ah, axis=1, keepdims=True)
        ebits = jax.lax.bitcast_convert_type(c, jnp.int32) >> 23
        fac = jax.lax.bitcast_convert_type((254 - ebits) << 23, jnp.float32)
        return ah * fac, eacc + (ebits - 127)

    def substep(alpha, t):
        return jax.lax.dot_general(alpha, expA, dnm,
                                   preferred_element_type=jnp.float32) * eb_ref[t]

    alpha, eacc = renorm(pi * eb_ref[0], jnp.zeros((_B, 1), jnp.int32))

    def block(i, carry):
        alpha, eacc = carry
        t0 = 1 + i * _NB
        for j in range(_NB):
            alpha = substep(alpha, t0 + j)
        return renorm(alpha, eacc)

    alpha, eacc = jax.lax.fori_loop(0, _NBLK, block, (alpha, eacc))
    for t in range(_TAIL0, _L):
        alpha, eacc = renorm(substep(alpha, t), eacc)
    s = jnp.sum(alpha, axis=1, keepdims=True)
    out_ref[:] = m_ref[:] + eacc.astype(jnp.float32) * _LN2 + jnp.log(s)


@functools.partial(jax.jit, static_argnames=())
def kernel(X, log_A, log_pi, means, log_vars):
    xt = X.reshape(_B, _L, _D).swapaxes(0, 1).reshape(_T, _D)
    out = pl.pallas_call(
        _hmm_body,
        out_shape=jax.ShapeDtypeStruct((_B, 1), jnp.float32),
        scratch_shapes=[
            pltpu.VMEM((_L, _B, _K), jnp.float32),
            pltpu.VMEM((_B, 1), jnp.float32),
        ],
    )(xt, log_A, log_pi.reshape(1, _K), means, log_vars)
    return out.reshape(_B)


# speedup vs baseline: 7.0181x; 1.0051x over previous
import functools
import math

import jax
import jax.numpy as jnp
from jax.experimental import pallas as pl
from jax.experimental.pallas import tpu as pltpu

_B = 16
_L = 2048
_T = _B * _L
_D = 32
_K = 16
_LOG2PI = math.log(2.0 * math.pi)
_LN2 = math.log(2.0)
_NB = 4
_UNROLL = 4
_STEP = _NB * _UNROLL
_NBLK = (_L - 1) // _STEP
_TAIL0 = 1 + _NBLK * _STEP


def _hmm_body(xt_ref, logA_ref, logpi_ref, means_ref, logvars_ref,
              out_ref, eb_ref, m_ref):
    x = xt_ref[:]
    lv = logvars_ref[:]
    iv = jnp.exp(-lv)
    w = means_ref[:] * iv
    cst = jnp.sum(means_ref[:] * w + lv, axis=1, keepdims=True) + _D * _LOG2PI
    dn = (((1,), (1,)), ((), ()))
    q = (jax.lax.dot_general(x * x, iv, dn, preferred_element_type=jnp.float32)
         - 2.0 * jax.lax.dot_general(x, w, dn, preferred_element_type=jnp.float32))
    logb = (-0.5 * (q + cst.reshape(1, _K))).reshape(_L, _B, _K)
    m3 = jnp.max(logb, axis=2, keepdims=True)
    eb_ref[:] = jnp.exp(logb - m3)
    m_ref[:] = jnp.sum(m3, axis=0)

    expA = jnp.exp(logA_ref[:])
    pi = jnp.exp(logpi_ref[:])
    dnm = (((1,), (0,)), ((), ()))

    def renorm(ah, eacc):
        c = jnp.sum(ah, axis=1, keepdims=True)
        ebits = jax.lax.bitcast_convert_type(c, jnp.int32) >> 23
        fac = jax.lax.bitcast_convert_type((254 - ebits) << 23, jnp.float32)
        return ah * fac, eacc + (ebits - 127)

    def substep(alpha, t):
        return jax.lax.dot_general(alpha, expA, dnm,
                                   preferred_element_type=jnp.float32) * eb_ref[t]

    alpha, eacc = renorm(pi * eb_ref[0], jnp.zeros((_B, 1), jnp.int32))

    def block(i, carry):
        alpha, eacc = carry
        t0 = 1 + i * _STEP
        for u in range(_UNROLL):
            for j in range(_NB):
                alpha = substep(alpha, t0 + u * _NB + j)
            alpha, eacc = renorm(alpha, eacc)
        return alpha, eacc

    alpha, eacc = jax.lax.fori_loop(0, _NBLK, block, (alpha, eacc))
    for t in range(_TAIL0, _L):
        alpha, eacc = renorm(substep(alpha, t), eacc)
    s = jnp.sum(alpha, axis=1, keepdims=True)
    out_ref[:] = m_ref[:] + eacc.astype(jnp.float32) * _LN2 + jnp.log(s)


@functools.partial(jax.jit, static_argnames=())
def kernel(X, log_A, log_pi, means, log_vars):
    xt = X.reshape(_B, _L, _D).swapaxes(0, 1).reshape(_T, _D)
    out = pl.pallas_call(
        _hmm_body,
        out_shape=jax.ShapeDtypeStruct((_B, 1), jnp.float32),
        scratch_shapes=[
            pltpu.VMEM((_L, _B, _K), jnp.float32),
            pltpu.VMEM((_B, 1), jnp.float32),
        ],
    )(xt, log_A, log_pi.reshape(1, _K), means, log_vars)
    return out.reshape(_B)


# speedup vs baseline: 7.0442x; 1.0037x over previous
import functools
import math

import jax
import jax.numpy as jnp
from jax.experimental import pallas as pl
from jax.experimental.pallas import tpu as pltpu

_B = 16
_L = 2048
_T = _B * _L
_D = 32
_K = 16
_LOG2PI = math.log(2.0 * math.pi)
_LN2 = math.log(2.0)
_G = 4
_NB = 4
_UNROLL = 4
_STEP = _NB * _UNROLL
_NBLK = (_L - 1) // _STEP
_TAIL0 = 1 + _NBLK * _STEP


def _hmm_body(xt_ref, logA_ref, logpi_ref, means_ref, logvars_ref,
              out_ref, eb_ref, m_ref):
    x = xt_ref[:]
    lv = logvars_ref[:]
    iv = jnp.exp(-lv)
    w = means_ref[:] * iv
    cst = jnp.sum(means_ref[:] * w + lv, axis=1, keepdims=True) + _D * _LOG2PI
    dn = (((1,), (1,)), ((), ()))
    q = (jax.lax.dot_general(x * x, iv, dn, preferred_element_type=jnp.float32)
         - 2.0 * jax.lax.dot_general(x, w, dn, preferred_element_type=jnp.float32))
    logb = (-0.5 * (q + cst.reshape(1, _K))).reshape(_L, _B, _K)
    m3 = jnp.max(logb, axis=2, keepdims=True)
    eb_ref[:] = jnp.exp(logb - m3)
    m_ref[:] = jnp.sum(m3, axis=0)

    expA = jnp.exp(logA_ref[:])
    pi = jnp.exp(logpi_ref[:])
    dnm = (((1,), (0,)), ((), ()))

    def renorm(ah, eacc):
        c = jnp.sum(ah, axis=1, keepdims=True)
        ebits = jax.lax.bitcast_convert_type(c, jnp.int32) >> 23
        fac = jax.lax.bitcast_convert_type((254 - ebits) << 23, jnp.float32)
        return ah * fac, eacc + (ebits - 127)

    gsz = _B // _G
    a0, e0 = renorm(pi * eb_ref[0], jnp.zeros((_B, 1), jnp.int32))
    alphas = tuple(a0[g * gsz:(g + 1) * gsz] for g in range(_G))
    eaccs = tuple(e0[g * gsz:(g + 1) * gsz] for g in range(_G))

    def block(i, carry):
        alphas, eaccs = carry
        t0 = 1 + i * _STEP
        for u in range(_UNROLL):
            for j in range(_NB):
                ebt = eb_ref[t0 + u * _NB + j]
                alphas = tuple(
                    jax.lax.dot_general(alphas[g], expA, dnm,
                                        preferred_element_type=jnp.float32)
                    * ebt[g * gsz:(g + 1) * gsz]
                    for g in range(_G))
            pairs = tuple(renorm(alphas[g], eaccs[g]) for g in range(_G))
            alphas = tuple(p[0] for p in pairs)
            eaccs = tuple(p[1] for p in pairs)
        return alphas, eaccs

    alphas, eaccs = jax.lax.fori_loop(0, _NBLK, block, (alphas, eaccs))
    for t in range(_TAIL0, _L):
        ebt = eb_ref[t]
        pairs = tuple(
            renorm(jax.lax.dot_general(alphas[g], expA, dnm,
                                       preferred_element_type=jnp.float32)
                   * ebt[g * gsz:(g + 1) * gsz], eaccs[g])
            for g in range(_G))
        alphas = tuple(p[0] for p in pairs)
        eaccs = tuple(p[1] for p in pairs)
    alpha = jnp.concatenate(alphas, axis=0)
    eacc = jnp.concatenate(eaccs, axis=0)
    s = jnp.sum(alpha, axis=1, keepdims=True)
    out_ref[:] = m_ref[:] + eacc.astype(jnp.float32) * _LN2 + jnp.log(s)


@functools.partial(jax.jit, static_argnames=())
def kernel(X, log_A, log_pi, means, log_vars):
    xt = X.reshape(_B, _L, _D).swapaxes(0, 1).reshape(_T, _D)
    out = pl.pallas_call(
        _hmm_body,
        out_shape=jax.ShapeDtypeStruct((_B, 1), jnp.float32),
        scratch_shapes=[
            pltpu.VMEM((_L, _B, _K), jnp.float32),
            pltpu.VMEM((_B, 1), jnp.float32),
        ],
    )(xt, log_A, log_pi.reshape(1, _K), means, log_vars)
    return out.reshape(_B)


# speedup vs baseline: 7.0615x; 1.0025x over previous
import functools
import math

import jax
import jax.numpy as jnp
from jax.experimental import pallas as pl
from jax.experimental.pallas import tpu as pltpu

_B = 16
_L = 2048
_T = _B * _L
_D = 32
_K = 16
_LOG2PI = math.log(2.0 * math.pi)
_LN2 = math.log(2.0)
_G = 4
_NB = 4
_UNROLL = 4
_STEP = _NB * _UNROLL
_NBLK = (_L - 1) // _STEP
_TAIL0 = 1 + _NBLK * _STEP


def _hmm_body(xt_ref, logA_ref, logpi_ref, means_ref, logvars_ref,
              out_ref, eb_ref, m_ref):
    x = xt_ref[:]
    lv = logvars_ref[:]
    iv = jnp.exp(-lv)
    w = means_ref[:] * iv
    cst = jnp.sum(means_ref[:] * w + lv, axis=1, keepdims=True) + _D * _LOG2PI
    dn = (((1,), (1,)), ((), ()))
    q = (jax.lax.dot_general(x * x, iv, dn, preferred_element_type=jnp.float32)
         - 2.0 * jax.lax.dot_general(x, w, dn, preferred_element_type=jnp.float32))
    logb = (-0.5 * (q + cst.reshape(1, _K))).reshape(_L, _B, _K)
    m3 = jnp.max(logb, axis=2, keepdims=True)
    eb_ref[:] = jnp.exp(logb - m3)
    m_ref[:] = jnp.sum(m3, axis=0)

    expA = jnp.exp(logA_ref[:])
    pi = jnp.exp(logpi_ref[:])
    dnm = (((1,), (0,)), ((), ()))

    def renorm(ah, eacc):
        c = jnp.sum(ah, axis=1, keepdims=True)
        ebits = jax.lax.bitcast_convert_type(c, jnp.int32) >> 23
        fac = jax.lax.bitcast_convert_type((254 - ebits) << 23, jnp.float32)
        return ah * fac, eacc + (ebits - 127)

    gsz = _B // _G
    a0, e0 = renorm(pi * eb_ref[0], jnp.zeros((_B, 1), jnp.int32))
    alphas = tuple(a0[g * gsz:(g + 1) * gsz] for g in range(_G))
    eaccs = tuple(e0[g * gsz:(g + 1) * gsz] for g in range(_G))

    def block(i, carry):
        alphas, eaccs = carry
        t0 = 1 + i * _STEP
        ebblk = eb_ref[pl.ds(t0, _STEP)]
        for u in range(_UNROLL):
            for j in range(_NB):
                ebt = ebblk[u * _NB + j]
                alphas = tuple(
                    jax.lax.dot_general(alphas[g], expA, dnm,
                                        preferred_element_type=jnp.float32)
                    * ebt[g * gsz:(g + 1) * gsz]
                    for g in range(_G))
            pairs = tuple(renorm(alphas[g], eaccs[g]) for g in range(_G))
            alphas = tuple(p[0] for p in pairs)
            eaccs = tuple(p[1] for p in pairs)
        return alphas, eaccs

    alphas, eaccs = jax.lax.fori_loop(0, _NBLK, block, (alphas, eaccs))
    for t in range(_TAIL0, _L):
        ebt = eb_ref[t]
        pairs = tuple(
            renorm(jax.lax.dot_general(alphas[g], expA, dnm,
                                       preferred_element_type=jnp.float32)
                   * ebt[g * gsz:(g + 1) * gsz], eaccs[g])
            for g in range(_G))
        alphas = tuple(p[0] for p in pairs)
        eaccs = tuple(p[1] for p in pairs)
    alpha = jnp.concatenate(alphas, axis=0)
    eacc = jnp.concatenate(eaccs, axis=0)
    s = jnp.sum(alpha, axis=1, keepdims=True)
    out_ref[:] = m_ref[:] + eacc.astype(jnp.float32) * _LN2 + jnp.log(s)


@functools.partial(jax.jit, static_argnames=())
def kernel(X, log_A, log_pi, means, log_vars):
    xt = X.reshape(_B, _L, _D).swapaxes(0, 1).reshape(_T, _D)
    out = pl.pallas_call(
        _hmm_body,
        out_shape=jax.ShapeDtypeStruct((_B, 1), jnp.float32),
        scratch_shapes=[
            pltpu.VMEM((_L, _B, _K), jnp.float32),
            pltpu.VMEM((_B, 1), jnp.float32),
        ],
    )(xt, log_A, log_pi.reshape(1, _K), means, log_vars)
    return out.reshape(_B)


# speedup vs baseline: 7.1606x; 1.0140x over previous
import functools
import math

import jax
import jax.numpy as jnp
from jax.experimental import pallas as pl
from jax.experimental.pallas import tpu as pltpu

_B = 16
_L = 2048
_T = _B * _L
_D = 32
_K = 16
_LOG2PI = math.log(2.0 * math.pi)
_LN2 = math.log(2.0)
_G = 4
_NB = 4
_UNROLL = 4
_STEP = _NB * _UNROLL
_NBLK = (_L - 1) // _STEP
_TAIL0 = 1 + _NBLK * _STEP


def _hmm_body(xt_ref, logA_ref, logpi_ref, means_ref, logvars_ref,
              out_ref, eb_ref, m_ref):
    x = xt_ref[:]
    lv = logvars_ref[:]
    iv = jnp.exp(-lv)
    w = means_ref[:] * iv
    cst = jnp.sum(means_ref[:] * w + lv, axis=1, keepdims=True) + _D * _LOG2PI
    dn = (((1,), (1,)), ((), ()))
    q = (jax.lax.dot_general(x * x, iv, dn, preferred_element_type=jnp.float32)
         - 2.0 * jax.lax.dot_general(x, w, dn, preferred_element_type=jnp.float32))
    logb = (-0.5 * (q + cst.reshape(1, _K))).reshape(_B, _L, _K)
    m3 = jnp.max(logb, axis=2, keepdims=True)
    eb_ref[:] = jnp.exp(logb - m3)
    m_ref[:] = jnp.sum(m3, axis=1)

    expA = jnp.exp(logA_ref[:])
    pi = jnp.exp(logpi_ref[:])
    dnm = (((1,), (0,)), ((), ()))

    def renorm(ah, eacc):
        c = jnp.sum(ah, axis=1, keepdims=True)
        ebits = jax.lax.bitcast_convert_type(c, jnp.int32) >> 23
        fac = jax.lax.bitcast_convert_type((254 - ebits) << 23, jnp.float32)
        return ah * fac, eacc + (ebits - 127)

    gsz = _B // _G
    a0, e0 = renorm(pi * eb_ref[:, 0], jnp.zeros((_B, 1), jnp.int32))
    alphas = tuple(a0[g * gsz:(g + 1) * gsz] for g in range(_G))
    eaccs = tuple(e0[g * gsz:(g + 1) * gsz] for g in range(_G))

    def block(i, carry):
        alphas, eaccs = carry
        t0 = 1 + i * _STEP
        ebblk = eb_ref[:, pl.ds(t0, _STEP)]
        for u in range(_UNROLL):
            for j in range(_NB):
                ebt = ebblk[:, u * _NB + j]
                alphas = tuple(
                    jax.lax.dot_general(alphas[g], expA, dnm,
                                        preferred_element_type=jnp.float32)
                    * ebt[g * gsz:(g + 1) * gsz]
                    for g in range(_G))
            pairs = tuple(renorm(alphas[g], eaccs[g]) for g in range(_G))
            alphas = tuple(p[0] for p in pairs)
            eaccs = tuple(p[1] for p in pairs)
        return alphas, eaccs

    alphas, eaccs = jax.lax.fori_loop(0, _NBLK, block, (alphas, eaccs))
    for t in range(_TAIL0, _L):
        ebt = eb_ref[:, t]
        pairs = tuple(
            renorm(jax.lax.dot_general(alphas[g], expA, dnm,
                                       preferred_element_type=jnp.float32)
                   * ebt[g * gsz:(g + 1) * gsz], eaccs[g])
            for g in range(_G))
        alphas = tuple(p[0] for p in pairs)
        eaccs = tuple(p[1] for p in pairs)
    alpha = jnp.concatenate(alphas, axis=0)
    eacc = jnp.concatenate(eaccs, axis=0)
    s = jnp.sum(alpha, axis=1, keepdims=True)
    out_ref[:] = m_ref[:] + eacc.astype(jnp.float32) * _LN2 + jnp.log(s)


@functools.partial(jax.jit, static_argnames=())
def kernel(X, log_A, log_pi, means, log_vars):
    out = pl.pallas_call(
        _hmm_body,
        out_shape=jax.ShapeDtypeStruct((_B, 1), jnp.float32),
        scratch_shapes=[
            pltpu.VMEM((_B, _L, _K), jnp.float32),
            pltpu.VMEM((_B, 1), jnp.float32),
        ],
    )(X, log_A, log_pi.reshape(1, _K), means, log_vars)
    return out.reshape(_B)


# speedup vs baseline: 19.5435x; 2.7293x over previous
import functools
import math

import jax
import jax.numpy as jnp
from jax import lax
from jax.experimental import pallas as pl
from jax.experimental.pallas import tpu as pltpu
from jax.experimental.pallas import tpu_sc as plsc

_B = 16
_L = 2048
_T = _B * _L
_D = 32
_K = 16
_LOG2PI = math.log(2.0 * math.pi)
_LN2 = math.log(2.0)
_NB = 4
_RPW = _L // 8


def _emit_body(x_ref, logA_ref, logpi_ref, means_ref, logvars_ref,
               eb_ref, msum_ref):
    x = x_ref[:]
    lv = logvars_ref[:]
    iv = jnp.exp(-lv)
    w = means_ref[:] * iv
    cst = jnp.sum(means_ref[:] * w + lv, axis=1, keepdims=True) + _D * _LOG2PI
    dn = (((1,), (1,)), ((), ()))
    q = (jax.lax.dot_general(x * x, iv, dn, preferred_element_type=jnp.float32)
         - 2.0 * jax.lax.dot_general(x, w, dn, preferred_element_type=jnp.float32))
    logb = (-0.5 * (q + cst.reshape(1, _K))).reshape(_B, _L, _K)
    m3 = jnp.max(logb, axis=2, keepdims=True)
    eb_ref[:] = jnp.exp(logb - m3).reshape(_T, _K)
    msum_ref[:] = jnp.sum(m3, axis=1)


_GDN = lax.GatherDimensionNumbers(offset_dims=(), collapsed_slice_dims=(0,),
                                  start_index_map=(0,))


def _lanes(a, idx):
    return lax.gather(a, idx[:, None], _GDN, (1,),
                      mode=lax.GatherScatterMode.PROMISE_IN_BOUNDS)


def _scan_body(eb_hbm, aux_hbm, alpha_out, e_out, ebv, auxv, aov, eov):
    wid = lax.axis_index("s") * 2 + lax.axis_index("c")

    @pl.when(wid < _B)
    def _():
        pltpu.sync_copy(eb_hbm.at[pl.ds(wid * _RPW, _RPW)], ebv)
        pltpu.sync_copy(aux_hbm, auxv)
        pi = jnp.exp(auxv[0, 0:_K])
        arows = [jnp.exp(auxv[(_K + _K * i) // 128,
                              pl.ds(((_K + _K * i) % 128), _K)])
                 for i in range(_K)]
        bidx = [jnp.full((_K,), i, jnp.int32) for i in range(_K)]
        lane = lax.iota(jnp.int32, _K)
        fly = [lane ^ d for d in (8, 4, 2, 1)]

        def matvec(a):
            terms = [arows[i] * _lanes(a, bidx[i]) for i in range(_K)]
            while len(terms) > 1:
                terms = [terms[2 * i] + terms[2 * i + 1]
                         for i in range(len(terms) // 2)]
            return terms[0]

        def renorm(a, ev):
            cv = a
            for f in fly:
                cv = cv + _lanes(cv, f)
            fac = jnp.full((_K,), 1.0, jnp.float32)
            boost = jnp.zeros((_K,), jnp.float32)
            for k in (64, 32, 16, 8, 4, 2, 1):
                cond = (cv * fac) < (2.0 ** (-k))
                fac = jnp.where(cond, fac * (2.0 ** k), fac)
                boost = jnp.where(cond, boost + float(k), boost)
            return a * fac, ev - boost

        zero = jnp.zeros((_K,), jnp.float32)
        alpha, ev = renorm(pi * ebv[0, 0:_K], zero)
        for t in range(1, 8):
            alpha = matvec(alpha) * ebv[0, pl.ds(t * _K, _K)]
            if t in (4, 7):
                alpha, ev = renorm(alpha, ev)

        def row_block(r, carry):
            alpha, ev = carry
            for j in range(8):
                alpha = matvec(alpha) * ebv[r, pl.ds(j * _K, _K)]
                if j in (3, 7):
                    alpha, ev = renorm(alpha, ev)
            return alpha, ev

        alpha, ev = lax.fori_loop(1, _RPW, row_block, (alpha, ev))
        for i in range(8):
            aov[pl.ds(i * _K, _K)] = alpha if i == 0 else zero
            eov[pl.ds(i * _K, _K)] = ev if i == 0 else zero
        pltpu.sync_copy(aov, alpha_out.at[wid])
        pltpu.sync_copy(eov, e_out.at[wid])


@functools.partial(jax.jit, static_argnames=())
def kernel(X, log_A, log_pi, means, log_vars):
    eb, msum = pl.pallas_call(
        _emit_body,
        out_shape=[
            jax.ShapeDtypeStruct((_T, _K), jnp.float32),
            jax.ShapeDtypeStruct((_B, 1), jnp.float32),
        ],
    )(X, log_A, log_pi.reshape(1, _K), means, log_vars)
    eb = eb.reshape(_B * _RPW, 8 * _K)

    aux = jnp.concatenate(
        [log_pi, log_A.reshape(_K * _K), jnp.zeros((112,), jnp.float32)]
    ).reshape(3, 128)

    mesh = plsc.VectorSubcoreMesh(core_axis_name="c", subcore_axis_name="s")
    alpha_rows, e_rows = pl.kernel(
        _scan_body,
        out_type=[
            jax.ShapeDtypeStruct((_B, 128), jnp.float32),
            jax.ShapeDtypeStruct((_B, 128), jnp.float32),
        ],
        mesh=mesh,
        scratch_types=[
            pltpu.VMEM((_RPW, 8 * _K), jnp.float32),
            pltpu.VMEM((3, 128), jnp.float32),
            pltpu.VMEM((128,), jnp.float32),
            pltpu.VMEM((128,), jnp.float32),
        ],
    )(eb, aux)

    return (msum.reshape(_B) + _LN2 * e_rows[:, 0]
            + jnp.log(jnp.sum(alpha_rows[:, 0:_K], axis=1)))


# speedup vs baseline: 19.5519x; 1.0004x over previous
import functools
import math

import jax
import jax.numpy as jnp
from jax import lax
from jax.experimental import pallas as pl
from jax.experimental.pallas import tpu as pltpu
from jax.experimental.pallas import tpu_sc as plsc

_B = 16
_L = 2048
_T = _B * _L
_D = 32
_K = 16
_LOG2PI = math.log(2.0 * math.pi)
_LN2 = math.log(2.0)
_NB = 4
_RPW = _L // 8


def _emit_body(x_ref, logA_ref, logpi_ref, means_ref, logvars_ref,
               eb_ref, msum_ref):
    x = x_ref[:]
    lv = logvars_ref[:]
    iv = jnp.exp(-lv)
    w = means_ref[:] * iv
    cst = jnp.sum(means_ref[:] * w + lv, axis=1, keepdims=True) + _D * _LOG2PI
    dn = (((1,), (1,)), ((), ()))
    q = (jax.lax.dot_general(x * x, iv, dn, preferred_element_type=jnp.float32)
         - 2.0 * jax.lax.dot_general(x, w, dn, preferred_element_type=jnp.float32))
    logb = (-0.5 * (q + cst.reshape(1, _K))).reshape(_B, _L, _K)
    m3 = jnp.max(logb, axis=2, keepdims=True)
    eb_ref[:] = jnp.exp(logb - m3).reshape(_T, _K)
    msum_ref[:] = jnp.sum(m3, axis=1)


_GDN = lax.GatherDimensionNumbers(offset_dims=(), collapsed_slice_dims=(0,),
                                  start_index_map=(0,))


def _lanes(a, idx):
    return lax.gather(a, idx[:, None], _GDN, (1,),
                      mode=lax.GatherScatterMode.PROMISE_IN_BOUNDS)


def _scan_body(eb_hbm, aux_hbm, alpha_out, e_out, ebv, auxv, aov, eov):
    wid = lax.axis_index("s")

    @pl.when(lax.axis_index("c") == 0)
    def _():
        pltpu.sync_copy(eb_hbm.at[pl.ds(wid * _RPW, _RPW)], ebv)
        pltpu.sync_copy(aux_hbm, auxv)
        pi = jnp.exp(auxv[0, 0:_K])
        arows = [jnp.exp(auxv[(_K + _K * i) // 128,
                              pl.ds(((_K + _K * i) % 128), _K)])
                 for i in range(_K)]
        bidx = [jnp.full((_K,), i, jnp.int32) for i in range(_K)]
        lane = lax.iota(jnp.int32, _K)
        fly = [lane ^ d for d in (8, 4, 2, 1)]

        def matvec(a):
            terms = [arows[i] * _lanes(a, bidx[i]) for i in range(_K)]
            while len(terms) > 1:
                terms = [terms[2 * i] + terms[2 * i + 1]
                         for i in range(len(terms) // 2)]
            return terms[0]

        def renorm(a, ev):
            cv = a
            for f in fly:
                cv = cv + _lanes(cv, f)
            fac = jnp.full((_K,), 1.0, jnp.float32)
            boost = jnp.zeros((_K,), jnp.float32)
            for k in (64, 32, 16, 8, 4, 2, 1):
                cond = (cv * fac) < (2.0 ** (-k))
                fac = jnp.where(cond, fac * (2.0 ** k), fac)
                boost = jnp.where(cond, boost + float(k), boost)
            return a * fac, ev - boost

        zero = jnp.zeros((_K,), jnp.float32)
        alpha, ev = renorm(pi * ebv[0, 0:_K], zero)
        for t in range(1, 8):
            alpha = matvec(alpha) * ebv[0, pl.ds(t * _K, _K)]
            if t in (4, 7):
                alpha, ev = renorm(alpha, ev)

        def row_block(r, carry):
            alpha, ev = carry
            for j in range(8):
                alpha = matvec(alpha) * ebv[r, pl.ds(j * _K, _K)]
                if j in (3, 7):
                    alpha, ev = renorm(alpha, ev)
            return alpha, ev

        alpha, ev = lax.fori_loop(1, _RPW, row_block, (alpha, ev))
        for i in range(8):
            aov[pl.ds(i * _K, _K)] = alpha if i == 0 else zero
            eov[pl.ds(i * _K, _K)] = ev if i == 0 else zero
        pltpu.sync_copy(aov, alpha_out.at[wid])
        pltpu.sync_copy(eov, e_out.at[wid])


@functools.partial(jax.jit, static_argnames=())
def kernel(X, log_A, log_pi, means, log_vars):
    eb, msum = pl.pallas_call(
        _emit_body,
        out_shape=[
            jax.ShapeDtypeStruct((_T, _K), jnp.float32),
            jax.ShapeDtypeStruct((_B, 1), jnp.float32),
        ],
    )(X, log_A, log_pi.reshape(1, _K), means, log_vars)
    eb = eb.reshape(_B * _RPW, 8 * _K)

    aux = jnp.concatenate(
        [log_pi, log_A.reshape(_K * _K), jnp.zeros((112,), jnp.float32)]
    ).reshape(3, 128)

    mesh = plsc.VectorSubcoreMesh(core_axis_name="c", subcore_axis_name="s")
    alpha_rows, e_rows = pl.kernel(
        _scan_body,
        out_type=[
            jax.ShapeDtypeStruct((_B, 128), jnp.float32),
            jax.ShapeDtypeStruct((_B, 128), jnp.float32),
        ],
        mesh=mesh,
        scratch_types=[
            pltpu.VMEM((_RPW, 8 * _K), jnp.float32),
            pltpu.VMEM((3, 128), jnp.float32),
            pltpu.VMEM((128,), jnp.float32),
            pltpu.VMEM((128,), jnp.float32),
        ],
    )(eb, aux)

    return (msum.reshape(_B) + _LN2 * e_rows[:, 0]
            + jnp.log(jnp.sum(alpha_rows[:, 0:_K], axis=1)))


# speedup vs baseline: 21.7433x; 1.1121x over previous
import functools
import math

import jax
import jax.numpy as jnp
from jax import lax
from jax.experimental import pallas as pl
from jax.experimental.pallas import tpu as pltpu
from jax.experimental.pallas import tpu_sc as plsc

_B = 16
_L = 2048
_T = _B * _L
_D = 32
_K = 16
_LOG2PI = math.log(2.0 * math.pi)
_LN2 = math.log(2.0)
_NB = 4
_RPW = _L // 8


_GB = 8
_BC = _B // _GB
_TC_ = _BC * _L


def _emit_body(x_ref, logA_ref, logpi_ref, means_ref, logvars_ref,
               eb_ref, msum_ref):
    x = x_ref[:]
    lv = logvars_ref[:]
    iv = jnp.exp(-lv)
    w = means_ref[:] * iv
    cst = jnp.sum(means_ref[:] * w + lv, axis=1, keepdims=True) + _D * _LOG2PI
    dn = (((1,), (1,)), ((), ()))
    q = (jax.lax.dot_general(x * x, iv, dn, preferred_element_type=jnp.float32)
         - 2.0 * jax.lax.dot_general(x, w, dn, preferred_element_type=jnp.float32))
    logb = (-0.5 * (q + cst.reshape(1, _K))).reshape(_BC, _L, _K)
    m3 = jnp.max(logb, axis=2, keepdims=True)
    eb4 = jnp.exp(logb - m3).reshape(_BC, _L // 8, 8, _K)
    eb_ref[:] = jnp.concatenate(
        [eb4[:, :, a, :].reshape(_BC * (_L // 8), _K) for a in range(8)],
        axis=1)
    msum_ref[:] = jnp.sum(m3, axis=1).reshape(1, _BC, 1)


_GDN = lax.GatherDimensionNumbers(offset_dims=(), collapsed_slice_dims=(0,),
                                  start_index_map=(0,))


def _lanes(a, idx):
    return lax.gather(a, idx[:, None], _GDN, (1,),
                      mode=lax.GatherScatterMode.PROMISE_IN_BOUNDS)


def _scan_body(eb_hbm, aux_hbm, alpha_out, e_out, ebv, auxv, aov, eov):
    wid = lax.axis_index("s")

    @pl.when(lax.axis_index("c") == 0)
    def _():
        pltpu.sync_copy(eb_hbm.at[pl.ds(wid * _RPW, _RPW)], ebv)
        pltpu.sync_copy(aux_hbm, auxv)
        pi = jnp.exp(auxv[0, 0:_K])
        arows = [jnp.exp(auxv[(_K + _K * i) // 128,
                              pl.ds(((_K + _K * i) % 128), _K)])
                 for i in range(_K)]
        bidx = [jnp.full((_K,), i, jnp.int32) for i in range(_K)]
        lane = lax.iota(jnp.int32, _K)
        fly = [lane ^ d for d in (8, 4, 2, 1)]

        def matvec(a):
            terms = [arows[i] * _lanes(a, bidx[i]) for i in range(_K)]
            while len(terms) > 1:
                terms = [terms[2 * i] + terms[2 * i + 1]
                         for i in range(len(terms) // 2)]
            return terms[0]

        def renorm(a, ev):
            cv = a
            for f in fly:
                cv = cv + _lanes(cv, f)
            fac = jnp.full((_K,), 1.0, jnp.float32)
            boost = jnp.zeros((_K,), jnp.float32)
            for k in (64, 32, 16, 8, 4, 2, 1):
                cond = (cv * fac) < (2.0 ** (-k))
                fac = jnp.where(cond, fac * (2.0 ** k), fac)
                boost = jnp.where(cond, boost + float(k), boost)
            return a * fac, ev - boost

        zero = jnp.zeros((_K,), jnp.float32)
        alpha, ev = renorm(pi * ebv[0, 0:_K], zero)
        for t in range(1, 8):
            alpha = matvec(alpha) * ebv[0, pl.ds(t * _K, _K)]
            if t in (4, 7):
                alpha, ev = renorm(alpha, ev)

        def row_block(r, carry):
            alpha, ev = carry
            for j in range(8):
                alpha = matvec(alpha) * ebv[r, pl.ds(j * _K, _K)]
                if j in (3, 7):
                    alpha, ev = renorm(alpha, ev)
            return alpha, ev

        alpha, ev = lax.fori_loop(1, _RPW, row_block, (alpha, ev))
        for i in range(8):
            aov[pl.ds(i * _K, _K)] = alpha if i == 0 else zero
            eov[pl.ds(i * _K, _K)] = ev if i == 0 else zero
        pltpu.sync_copy(aov, alpha_out.at[wid])
        pltpu.sync_copy(eov, e_out.at[wid])


@functools.partial(jax.jit, static_argnames=())
def kernel(X, log_A, log_pi, means, log_vars):
    eb, msum = pl.pallas_call(
        _emit_body,
        grid=(_GB,),
        in_specs=[
            pl.BlockSpec((_TC_, _D), lambda i: (i, 0)),
            pl.BlockSpec((_K, _K), lambda i: (0, 0)),
            pl.BlockSpec((1, _K), lambda i: (0, 0)),
            pl.BlockSpec((_K, _D), lambda i: (0, 0)),
            pl.BlockSpec((_K, _D), lambda i: (0, 0)),
        ],
        out_specs=[
            pl.BlockSpec((_BC * _RPW, 8 * _K), lambda i: (i, 0)),
            pl.BlockSpec((1, _BC, 1), lambda i: (i, 0, 0)),
        ],
        out_shape=[
            jax.ShapeDtypeStruct((_B * _RPW, 8 * _K), jnp.float32),
            jax.ShapeDtypeStruct((_GB, _BC, 1), jnp.float32),
        ],
    )(X, log_A, log_pi.reshape(1, _K), means, log_vars)

    aux = jnp.concatenate(
        [log_pi, log_A.reshape(_K * _K), jnp.zeros((112,), jnp.float32)]
    ).reshape(3, 128)

    mesh = plsc.VectorSubcoreMesh(core_axis_name="c", subcore_axis_name="s")
    alpha_rows, e_rows = pl.kernel(
        _scan_body,
        out_type=[
            jax.ShapeDtypeStruct((_B, 128), jnp.float32),
            jax.ShapeDtypeStruct((_B, 128), jnp.float32),
        ],
        mesh=mesh,
        scratch_types=[
            pltpu.VMEM((_RPW, 8 * _K), jnp.float32),
            pltpu.VMEM((3, 128), jnp.float32),
            pltpu.VMEM((128,), jnp.float32),
            pltpu.VMEM((128,), jnp.float32),
        ],
    )(eb, aux)

    return (msum.reshape(_B) + _LN2 * e_rows[:, 0]
            + jnp.log(jnp.sum(alpha_rows[:, 0:_K], axis=1)))
